# baseline (device time: 42917 ns/iter reference)
import jax
import jax.numpy as jnp
from jax import lax
from jax.experimental import pallas as pl
from jax.experimental.pallas import tpu as pltpu


def kernel(dy, W):
    m, k = dy.shape
    d, _ = W.shape
    HALF = m // 2
    Q = m // 4

    my_y_out = lax.axis_index("y")
    dy_half = lax.dynamic_slice_in_dim(dy, my_y_out * HALF, HALF, 0).astype(
        jnp.bfloat16
    )
    w_bf16 = W.astype(jnp.bfloat16)

    def body(dy_ref, w_ref, out_ref,
             p_ref, rs_ref, r_ref, ax_ref, ay_ref, dq_ref,
             send_sems, recv_sems):
        my_x = lax.axis_index("x")
        my_y = lax.axis_index("y")

        barrier_sem = pltpu.get_barrier_semaphore()
        pl.semaphore_signal(
            barrier_sem, inc=1,
            device_id=(1 - my_x, my_y), device_id_type=pl.DeviceIdType.MESH,
        )
        pl.semaphore_signal(
            barrier_sem, inc=1,
            device_id=(my_x, 1 - my_y), device_id_type=pl.DeviceIdType.MESH,
        )
        pl.semaphore_wait(barrier_sem, 2)

        dy_non = dy_ref[pl.ds((1 - my_x) * Q, Q), :]
        p_ref[pl.ds((1 - my_x) * Q, Q), :] = lax.dot_general(
            dy_non, w_ref[...], (((1,), (1,)), ((), ())),
            preferred_element_type=jnp.float32,
        ).astype(jnp.bfloat16)

        rdma1 = pltpu.make_async_remote_copy(
            src_ref=p_ref.at[pl.ds((1 - my_x) * Q, Q), :],
            dst_ref=rs_ref,
            send_sem=send_sems.at[0],
            recv_sem=recv_sems.at[0],
            device_id=(1 - my_x, my_y),
            device_id_type=pl.DeviceIdType.MESH,
        )
        rdma1.start()

        dy_own = dy_ref[pl.ds(my_x * Q, Q), :]
        p_own = lax.dot_general(
            dy_own, w_ref[...], (((1,), (1,)), ((), ())),
            preferred_element_type=jnp.float32,
        )

        rdma1.wait_recv()
        r_f32 = p_own + rs_ref[...].astype(jnp.float32)
        my_off = my_y * HALF + my_x * Q
        out_ref[pl.ds(my_off, Q), :] = r_f32
        r_ref[...] = r_f32.astype(jnp.bfloat16)

        rdma2 = pltpu.make_async_remote_copy(
            src_ref=r_ref,
            dst_ref=ax_ref,
            send_sem=send_sems.at[1],
            recv_sem=recv_sems.at[1],
            device_id=(1 - my_x, my_y),
            device_id_type=pl.DeviceIdType.MESH,
        )
        rdma3 = pltpu.make_async_remote_copy(
            src_ref=r_ref,
            dst_ref=ay_ref,
            send_sem=send_sems.at[2],
            recv_sem=recv_sems.at[2],
            device_id=(my_x, 1 - my_y),
            device_id_type=pl.DeviceIdType.MESH,
        )
        rdma2.start()
        rdma3.start()
        rdma2.wait_recv()

        rdma4 = pltpu.make_async_remote_copy(
            src_ref=ax_ref,
            dst_ref=dq_ref,
            send_sem=send_sems.at[3],
            recv_sem=recv_sems.at[3],
            device_id=(my_x, 1 - my_y),
            device_id_type=pl.DeviceIdType.MESH,
        )
        rdma4.start()

        xq_off = my_y * HALF + (1 - my_x) * Q
        out_ref[pl.ds(xq_off, Q), :] = ax_ref[...].astype(jnp.float32)
        rdma3.wait_recv()
        yq_off = (1 - my_y) * HALF + my_x * Q
        out_ref[pl.ds(yq_off, Q), :] = ay_ref[...].astype(jnp.float32)
        rdma4.wait_recv()
        dq_off = (1 - my_y) * HALF + (1 - my_x) * Q
        out_ref[pl.ds(dq_off, Q), :] = dq_ref[...].astype(jnp.float32)

        rdma1.wait_send()
        rdma2.wait_send()
        rdma3.wait_send()
        rdma4.wait_send()

    return pl.pallas_call(
        body,
        out_shape=jax.ShapeDtypeStruct((m, d), jnp.float32),
        in_specs=[
            pl.BlockSpec(memory_space=pltpu.VMEM),
            pl.BlockSpec(memory_space=pltpu.VMEM),
        ],
        out_specs=pl.BlockSpec(memory_space=pltpu.VMEM),
        scratch_shapes=[
            pltpu.VMEM((HALF, d), jnp.bfloat16),
            pltpu.VMEM((Q, d), jnp.bfloat16),
            pltpu.VMEM((Q, d), jnp.bfloat16),
            pltpu.VMEM((Q, d), jnp.bfloat16),
            pltpu.VMEM((Q, d), jnp.bfloat16),
            pltpu.VMEM((Q, d), jnp.bfloat16),
            pltpu.SemaphoreType.DMA((4,)),
            pltpu.SemaphoreType.DMA((4,)),
        ],
        compiler_params=pltpu.CompilerParams(collective_id=0),
    )(dy_half, w_bf16)


# device time: 33256 ns/iter; 1.2905x vs baseline; 1.2905x over previous
import jax
import jax.numpy as jnp
from jax import lax
from jax.experimental import pallas as pl
from jax.experimental.pallas import tpu as pltpu

NC = 4


def kernel(dy, W):
    m, k = dy.shape
    d, _ = W.shape
    HALF = m // 2
    Q = m // 4
    CW = d // NC

    def body(dy_any, w_any, out_ref,
             dyv, dybf, wv, wbf, pn, po, rsb, rbf, axb, ayb, dqb,
             dy_sem, w_sems, rs_s, rs_r, ax_s, ax_r, ay_s, ay_r, b_s, b_r):
        my_x = lax.axis_index("x")
        my_y = lax.axis_index("y")
        x_nb = (1 - my_x, my_y)
        y_nb = (my_x, 1 - my_y)

        dy_dma = pltpu.make_async_copy(
            dy_any.at[pl.ds(my_y * HALF, HALF), :], dyv, dy_sem
        )
        dy_dma.start()

        def w_dma(c):
            return pltpu.make_async_copy(
                w_any.at[pl.ds(c * CW, CW), :], wv.at[c % 2], w_sems.at[c % 2]
            )

        w_dma(0).start()
        w_dma(1).start()

        barrier_sem = pltpu.get_barrier_semaphore()
        pl.semaphore_signal(
            barrier_sem, inc=1,
            device_id=x_nb, device_id_type=pl.DeviceIdType.MESH,
        )
        pl.semaphore_signal(
            barrier_sem, inc=1,
            device_id=y_nb, device_id_type=pl.DeviceIdType.MESH,
        )
        pl.semaphore_wait(barrier_sem, 2)

        dy_dma.wait()
        dybf[...] = dyv[...].astype(jnp.bfloat16)

        def remote(src, dst, ssem, rsem, dev):
            return pltpu.make_async_remote_copy(
                src_ref=src, dst_ref=dst, send_sem=ssem, recv_sem=rsem,
                device_id=dev, device_id_type=pl.DeviceIdType.MESH,
            )

        rs = {c: remote(pn.at[c], rsb.at[c], rs_s.at[c], rs_r.at[c], x_nb)
              for c in range(NC)}
        ax = {c: remote(rbf.at[c], axb.at[c], ax_s.at[c], ax_r.at[c], x_nb)
              for c in range(NC)}
        ay = {c: remote(rbf.at[c], ayb.at[c], ay_s.at[c], ay_r.at[c], y_nb)
              for c in range(NC)}
        fw = {c: remote(axb.at[c], dqb.at[c], b_s.at[c], b_r.at[c], y_nb)
              for c in range(NC)}

        my_off = my_y * HALF + my_x * Q
        xq_off = my_y * HALF + (1 - my_x) * Q
        yq_off = (1 - my_y) * HALF + my_x * Q
        dq_off = (1 - my_y) * HALF + (1 - my_x) * Q

        def compute_stage(c):
            w_dma(c).wait()
            wbf[c % 2] = wv[c % 2].astype(jnp.bfloat16)
            if c + 2 < NC:
                w_dma(c + 2).start()
            dy_non = dybf[pl.ds((1 - my_x) * Q, Q), :]
            pn[c] = lax.dot_general(
                dy_non, wbf[c % 2], (((1,), (1,)), ((), ())),
                preferred_element_type=jnp.float32,
            ).astype(jnp.bfloat16)
            rs[c].start()
            dy_own = dybf[pl.ds(my_x * Q, Q), :]
            po[c] = lax.dot_general(
                dy_own, wbf[c % 2], (((1,), (1,)), ((), ())),
                preferred_element_type=jnp.float32,
            )

        def rs_finish(c):
            rs[c].wait_recv()
            r32 = po[c] + rsb[c].astype(jnp.float32)
            out_ref[pl.ds(my_off, Q), pl.ds(c * CW, CW)] = r32
            rbf[c] = r32.astype(jnp.bfloat16)
            ax[c].start()
            ay[c].start()

        def ab_finish(c):
            ax[c].wait_recv()
            out_ref[pl.ds(xq_off, Q), pl.ds(c * CW, CW)] = (
                axb[c].astype(jnp.float32)
            )
            fw[c].start()
            ay[c].wait_recv()
            out_ref[pl.ds(yq_off, Q), pl.ds(c * CW, CW)] = (
                ayb[c].astype(jnp.float32)
            )

        def b_finish(c):
            fw[c].wait_recv()
            out_ref[pl.ds(dq_off, Q), pl.ds(c * CW, CW)] = (
                dqb[c].astype(jnp.float32)
            )

        for c in range(NC):
            compute_stage(c)
            if c >= 1:
                rs_finish(c - 1)
            if c >= 2:
                ab_finish(c - 2)
            if c >= 3:
                b_finish(c - 3)
        rs_finish(NC - 1)
        ab_finish(NC - 2)
        b_finish(NC - 3)
        ab_finish(NC - 1)
        b_finish(NC - 2)
        b_finish(NC - 1)

        for c in range(NC):
            rs[c].wait_send()
            ax[c].wait_send()
            ay[c].wait_send()
            fw[c].wait_send()

    return pl.pallas_call(
        body,
        out_shape=jax.ShapeDtypeStruct((m, d), jnp.float32),
        in_specs=[
            pl.BlockSpec(memory_space=pltpu.MemorySpace.HBM),
            pl.BlockSpec(memory_space=pltpu.MemorySpace.HBM),
        ],
        out_specs=pl.BlockSpec(memory_space=pltpu.VMEM),
        scratch_shapes=[
            pltpu.VMEM((HALF, k), jnp.float32),
            pltpu.VMEM((HALF, k), jnp.bfloat16),
            pltpu.VMEM((2, CW, k), jnp.float32),
            pltpu.VMEM((2, CW, k), jnp.bfloat16),
            pltpu.VMEM((NC, Q, CW), jnp.bfloat16),
            pltpu.VMEM((NC, Q, CW), jnp.float32),
            pltpu.VMEM((NC, Q, CW), jnp.bfloat16),
            pltpu.VMEM((NC, Q, CW), jnp.bfloat16),
            pltpu.VMEM((NC, Q, CW), jnp.bfloat16),
            pltpu.VMEM((NC, Q, CW), jnp.bfloat16),
            pltpu.VMEM((NC, Q, CW), jnp.bfloat16),
            pltpu.SemaphoreType.DMA,
            pltpu.SemaphoreType.DMA((2,)),
            pltpu.SemaphoreType.DMA((NC,)),
            pltpu.SemaphoreType.DMA((NC,)),
            pltpu.SemaphoreType.DMA((NC,)),
            pltpu.SemaphoreType.DMA((NC,)),
            pltpu.SemaphoreType.DMA((NC,)),
            pltpu.SemaphoreType.DMA((NC,)),
            pltpu.SemaphoreType.DMA((NC,)),
            pltpu.SemaphoreType.DMA((NC,)),
        ],
        compiler_params=pltpu.CompilerParams(collective_id=0),
    )(dy, W)
